# baseline (speedup 1.0000x reference)
import functools

import jax
import jax.numpy as jnp
from jax import lax
from jax.experimental import pallas as pl
from jax.experimental.pallas import tpu as pltpu
from jax.experimental.pallas import tpu_sc as plsc

_N_F = 7
_N_P = 16
_COLS = _N_F + _N_P
_D = 64
_CHUNK = 128
_LANES = 16
_N_TILES = 32


def _table_body(f_ref, p_ref, wf_ref, wp_ref, b_ref, t_ref):
    wf = jnp.dot(f_ref[...], wf_ref[...], preferred_element_type=jnp.float32)
    wp = jnp.dot(p_ref[...], wp_ref[...], preferred_element_type=jnp.float32)
    t_ref[...] = wf[:, None, :] + wp[None, :, :] + b_ref[...]


def _build_table(f_table, p_table, fc_w, fc_b):
    t3 = pl.pallas_call(
        _table_body,
        out_shape=jax.ShapeDtypeStruct((_N_F, _N_P, _D), jnp.float32),
    )(
        f_table,
        p_table,
        fc_w[:, :_N_P].T,
        fc_w[:, _N_P:].T,
        fc_b.reshape(1, 1, _D),
    )
    return t3.reshape(_N_F * _N_P * _D)


def _edge_embed(edge, t_flat, n):
    assert n % _CHUNK == 0
    n_chunks = n // _CHUNK
    mesh = plsc.VectorSubcoreMesh(core_axis_name="c", subcore_axis_name="s")

    @functools.partial(
        pl.kernel,
        mesh=mesh,
        compiler_params=pltpu.CompilerParams(
            needs_layout_passes=False, use_tc_tiling_on_sc=True),
        out_type=jax.ShapeDtypeStruct((n, _D), jnp.float32),
        scratch_types=[
            pltpu.VMEM((_CHUNK, _COLS), jnp.float32),
            pltpu.VMEM((_CHUNK, _COLS), jnp.float32),
            pltpu.VMEM((_CHUNK, _D), jnp.float32),
            pltpu.VMEM((_CHUNK, _D), jnp.float32),
            pltpu.VMEM((_N_F * _N_P * _D,), jnp.float32),
            pltpu.SemaphoreType.DMA,
            pltpu.SemaphoreType.DMA,
            pltpu.SemaphoreType.DMA,
            pltpu.SemaphoreType.DMA,
        ],
    )
    def run(edge_hbm, t_hbm, out_hbm, ev0, ev1, ov0, ov1, t_v,
            si0, si1, so0, so1):
        cid = lax.axis_index("c")
        sid = lax.axis_index("s")
        wid = sid * 2 + cid
        pltpu.sync_copy(t_hbm, t_v)
        my_n = (n_chunks - wid + _N_TILES - 1) // _N_TILES
        edge_bufs = (ev0, ev1)
        out_bufs = (ov0, ov1)
        sin = (si0, si1)
        sout = (so0, so1)

        lane = lax.iota(jnp.int32, _LANES)
        fmask = lane < _N_F
        ninf = jnp.full((_LANES,), -jnp.inf, jnp.float32)

        def in_slice(i):
            return edge_hbm.at[pl.ds((wid + i * _N_TILES) * _CHUNK, _CHUNK)]

        def out_slice(i):
            return out_hbm.at[pl.ds((wid + i * _N_TILES) * _CHUNK, _CHUNK)]

        def compute(edge_v, out_v):
            def row_body(r, c2):
                va = edge_v[r, pl.ds(0, _LANES)]
                vb = edge_v[r, pl.ds(_N_F, _LANES)]
                va_f = jnp.where(fmask, va, ninf)
                feq = va_f == jnp.max(va_f)
                fidx = plsc.all_reduce_ffs(feq)
                peq = vb == jnp.max(vb)
                pidx = plsc.all_reduce_ffs(peq)
                tb = ((fidx * _N_P + pidx) * _D)[0]
                for q in range(0, _D, _LANES):
                    out_v[r, pl.ds(q, _LANES)] = t_v[pl.ds(tb + q, _LANES)]
                return c2

            lax.fori_loop(0, _CHUNK, row_body, 0, unroll=4)

        for b in range(2):
            @pl.when(b < my_n)
            def _():
                pltpu.async_copy(in_slice(b), edge_bufs[b], sin[b])

        def outer(j, carry):
            for b in range(2):
                i = 2 * j + b

                @pl.when(i < my_n)
                def _():
                    pltpu.make_async_copy(in_slice(i), edge_bufs[b], sin[b]).wait()

                    @pl.when(i >= 2)
                    def _():
                        pltpu.make_async_copy(
                            out_bufs[b], out_slice(i - 2), sout[b]).wait()

                    compute(edge_bufs[b], out_bufs[b])
                    pltpu.async_copy(out_bufs[b], out_slice(i), sout[b])

                    @pl.when(i + 2 < my_n)
                    def _():
                        pltpu.async_copy(in_slice(i + 2), edge_bufs[b], sin[b])
            return carry

        lax.fori_loop(0, (my_n + 1) // 2, outer, 0)

        for b in range(2):
            @pl.when(b < my_n)
            def _():
                pltpu.make_async_copy(out_bufs[b], out_slice(b), sout[b]).wait()

    return run(edge, t_flat)


def kernel(edge, p_table, f_table, fc_w, fc_b):
    n = edge.shape[0]
    t = _build_table(f_table, p_table, fc_w, fc_b)
    return _edge_embed(edge, t, n)


# speedup vs baseline: 1.7978x; 1.7978x over previous
import functools

import jax
import jax.numpy as jnp
from jax import lax
from jax.experimental import pallas as pl
from jax.experimental.pallas import tpu as pltpu
from jax.experimental.pallas import tpu_sc as plsc

_N_F = 7
_N_P = 16
_COLS = _N_F + _N_P
_D = 64
_CHUNK = 128
_LANES = 16
_N_TILES = 32


def _table_body(f_ref, p_ref, wf_ref, wp_ref, b_ref, t_ref):
    wf = jnp.dot(f_ref[...], wf_ref[...], preferred_element_type=jnp.float32)
    wp = jnp.dot(p_ref[...], wp_ref[...], preferred_element_type=jnp.float32)
    t_ref[...] = wf[:, None, :] + wp[None, :, :] + b_ref[...]


def _build_table(f_table, p_table, fc_w, fc_b):
    t3 = pl.pallas_call(
        _table_body,
        out_shape=jax.ShapeDtypeStruct((_N_F, _N_P, _D), jnp.float32),
    )(
        f_table,
        p_table,
        fc_w[:, :_N_P].T,
        fc_w[:, _N_P:].T,
        fc_b.reshape(1, 1, _D),
    )
    return t3.reshape(_N_F * _N_P * _D)


def _edge_embed(edge, t_flat, n):
    assert n % _CHUNK == 0
    n_chunks = n // _CHUNK
    mesh = plsc.VectorSubcoreMesh(core_axis_name="c", subcore_axis_name="s")

    @functools.partial(
        pl.kernel,
        mesh=mesh,
        compiler_params=pltpu.CompilerParams(
            needs_layout_passes=False, use_tc_tiling_on_sc=True),
        out_type=jax.ShapeDtypeStruct((n, _D), jnp.float32),
        scratch_types=[
            pltpu.VMEM((_CHUNK, _COLS), jnp.float32),
            pltpu.VMEM((_CHUNK, _COLS), jnp.float32),
            pltpu.VMEM((_CHUNK, _D), jnp.float32),
            pltpu.VMEM((_CHUNK, _D), jnp.float32),
            pltpu.VMEM((_N_F * _N_P * _D,), jnp.float32),
            pltpu.SemaphoreType.DMA,
            pltpu.SemaphoreType.DMA,
            pltpu.SemaphoreType.DMA,
            pltpu.SemaphoreType.DMA,
        ],
    )
    def run(edge_hbm, t_hbm, out_hbm, ev0, ev1, ov0, ov1, t_v,
            si0, si1, so0, so1):
        cid = lax.axis_index("c")
        sid = lax.axis_index("s")
        wid = sid * 2 + cid
        pltpu.sync_copy(t_hbm, t_v)
        my_n = (n_chunks - wid + _N_TILES - 1) // _N_TILES
        edge_bufs = (ev0, ev1)
        out_bufs = (ov0, ov1)
        sin = (si0, si1)
        sout = (so0, so1)

        lane = lax.iota(jnp.int32, _LANES)
        fmask = lane < _N_F
        ninf = jnp.full((_LANES,), -jnp.inf, jnp.float32)

        def in_slice(i):
            return edge_hbm.at[pl.ds((wid + i * _N_TILES) * _CHUNK, _CHUNK)]

        def out_slice(i):
            return out_hbm.at[pl.ds((wid + i * _N_TILES) * _CHUNK, _CHUNK)]

        def compute(edge_v, out_v):
            @plsc.parallel_loop(0, _CHUNK, unroll=8)
            def row_body(r):
                va = edge_v[r, pl.ds(0, _LANES)]
                vb = edge_v[r, pl.ds(_N_F, _LANES)]
                va_f = jnp.where(fmask, va, ninf)
                feq = va_f == jnp.max(va_f)
                fidx = plsc.all_reduce_ffs(feq)
                peq = vb == jnp.max(vb)
                pidx = plsc.all_reduce_ffs(peq)
                tb = ((fidx * _N_P + pidx) * _D)[0]
                for q in range(0, _D, _LANES):
                    out_v[r, pl.ds(q, _LANES)] = t_v[pl.ds(tb + q, _LANES)]

        for b in range(2):
            @pl.when(b < my_n)
            def _():
                pltpu.async_copy(in_slice(b), edge_bufs[b], sin[b])

        def outer(j, carry):
            for b in range(2):
                i = 2 * j + b

                @pl.when(i < my_n)
                def _():
                    pltpu.make_async_copy(in_slice(i), edge_bufs[b], sin[b]).wait()

                    @pl.when(i >= 2)
                    def _():
                        pltpu.make_async_copy(
                            out_bufs[b], out_slice(i - 2), sout[b]).wait()

                    compute(edge_bufs[b], out_bufs[b])
                    pltpu.async_copy(out_bufs[b], out_slice(i), sout[b])

                    @pl.when(i + 2 < my_n)
                    def _():
                        pltpu.async_copy(in_slice(i + 2), edge_bufs[b], sin[b])
            return carry

        lax.fori_loop(0, (my_n + 1) // 2, outer, 0)

        for b in range(2):
            @pl.when(b < my_n)
            def _():
                pltpu.make_async_copy(out_bufs[b], out_slice(b), sout[b]).wait()

    return run(edge, t_flat)


def kernel(edge, p_table, f_table, fc_w, fc_b):
    n = edge.shape[0]
    t = _build_table(f_table, p_table, fc_w, fc_b)
    return _edge_embed(edge, t, n)


# speedup vs baseline: 1.7995x; 1.0009x over previous
import functools

import jax
import jax.numpy as jnp
from jax import lax
from jax.experimental import pallas as pl
from jax.experimental.pallas import tpu as pltpu
from jax.experimental.pallas import tpu_sc as plsc

_N_F = 7
_N_P = 16
_COLS = _N_F + _N_P
_D = 64
_CHUNK = 160
_LANES = 16
_N_TILES = 32


def _table_body(f_ref, p_ref, wf_ref, wp_ref, b_ref, t_ref):
    wf = jnp.dot(f_ref[...], wf_ref[...], preferred_element_type=jnp.float32)
    wp = jnp.dot(p_ref[...], wp_ref[...], preferred_element_type=jnp.float32)
    t_ref[...] = wf[:, None, :] + wp[None, :, :] + b_ref[...]


def _build_table(f_table, p_table, fc_w, fc_b):
    t3 = pl.pallas_call(
        _table_body,
        out_shape=jax.ShapeDtypeStruct((_N_F, _N_P, _D), jnp.float32),
    )(
        f_table,
        p_table,
        fc_w[:, :_N_P].T,
        fc_w[:, _N_P:].T,
        fc_b.reshape(1, 1, _D),
    )
    return t3.reshape(_N_F * _N_P * _D)


def _edge_embed(edge, t_flat, n):
    assert n % _CHUNK == 0
    n_chunks = n // _CHUNK
    mesh = plsc.VectorSubcoreMesh(core_axis_name="c", subcore_axis_name="s")

    @functools.partial(
        pl.kernel,
        mesh=mesh,
        compiler_params=pltpu.CompilerParams(
            needs_layout_passes=False, use_tc_tiling_on_sc=True),
        out_type=jax.ShapeDtypeStruct((n, _D), jnp.float32),
        scratch_types=[
            pltpu.VMEM((_CHUNK, _COLS), jnp.float32),
            pltpu.VMEM((_CHUNK, _COLS), jnp.float32),
            pltpu.VMEM((_CHUNK, _D), jnp.float32),
            pltpu.VMEM((_CHUNK, _D), jnp.float32),
            pltpu.VMEM((_N_F * _N_P * _D,), jnp.float32),
            pltpu.SemaphoreType.DMA,
            pltpu.SemaphoreType.DMA,
            pltpu.SemaphoreType.DMA,
            pltpu.SemaphoreType.DMA,
        ],
    )
    def run(edge_hbm, t_hbm, out_hbm, ev0, ev1, ov0, ov1, t_v,
            si0, si1, so0, so1):
        cid = lax.axis_index("c")
        sid = lax.axis_index("s")
        wid = sid * 2 + cid
        pltpu.sync_copy(t_hbm, t_v)
        my_n = (n_chunks - wid + _N_TILES - 1) // _N_TILES
        edge_bufs = (ev0, ev1)
        out_bufs = (ov0, ov1)
        sin = (si0, si1)
        sout = (so0, so1)

        lane = lax.iota(jnp.int32, _LANES)
        fmask = lane < _N_F
        ninf = jnp.full((_LANES,), -jnp.inf, jnp.float32)

        def in_slice(i):
            return edge_hbm.at[pl.ds((wid + i * _N_TILES) * _CHUNK, _CHUNK)]

        def out_slice(i):
            return out_hbm.at[pl.ds((wid + i * _N_TILES) * _CHUNK, _CHUNK)]

        def compute(edge_v, out_v):
            @plsc.parallel_loop(0, _CHUNK, unroll=8)
            def row_body(r):
                va = edge_v[r, pl.ds(0, _LANES)]
                vb = edge_v[r, pl.ds(_N_F, _LANES)]
                va_f = jnp.where(fmask, va, ninf)
                feq = va_f == jnp.max(va_f)
                fidx = plsc.all_reduce_ffs(feq)
                peq = vb == jnp.max(vb)
                pidx = plsc.all_reduce_ffs(peq)
                tb = ((fidx * _N_P + pidx) * _D)[0]
                for q in range(0, _D, _LANES):
                    out_v[r, pl.ds(q, _LANES)] = t_v[pl.ds(tb + q, _LANES)]

        for b in range(2):
            @pl.when(b < my_n)
            def _():
                pltpu.async_copy(in_slice(b), edge_bufs[b], sin[b])

        def outer(j, carry):
            for b in range(2):
                i = 2 * j + b

                @pl.when(i < my_n)
                def _():
                    pltpu.make_async_copy(in_slice(i), edge_bufs[b], sin[b]).wait()

                    @pl.when(i >= 2)
                    def _():
                        pltpu.make_async_copy(
                            out_bufs[b], out_slice(i - 2), sout[b]).wait()

                    compute(edge_bufs[b], out_bufs[b])
                    pltpu.async_copy(out_bufs[b], out_slice(i), sout[b])

                    @pl.when(i + 2 < my_n)
                    def _():
                        pltpu.async_copy(in_slice(i + 2), edge_bufs[b], sin[b])
            return carry

        lax.fori_loop(0, (my_n + 1) // 2, outer, 0)

        for b in range(2):
            @pl.when(b < my_n)
            def _():
                pltpu.make_async_copy(out_bufs[b], out_slice(b), sout[b]).wait()

    return run(edge, t_flat)


def kernel(edge, p_table, f_table, fc_w, fc_b):
    n = edge.shape[0]
    t = _build_table(f_table, p_table, fc_w, fc_b)
    return _edge_embed(edge, t, n)


# speedup vs baseline: 1.8018x; 1.0013x over previous
import functools

import jax
import jax.numpy as jnp
from jax import lax
from jax.experimental import pallas as pl
from jax.experimental.pallas import tpu as pltpu
from jax.experimental.pallas import tpu_sc as plsc

_N_F = 7
_N_P = 16
_COLS = _N_F + _N_P
_D = 64
_CHUNK = 128
_LANES = 16
_N_TILES = 32


def _table_body(f_ref, p_ref, wf_ref, wp_ref, b_ref, t_ref):
    wf = jnp.dot(f_ref[...], wf_ref[...], preferred_element_type=jnp.float32)
    wp = jnp.dot(p_ref[...], wp_ref[...], preferred_element_type=jnp.float32)
    t_ref[...] = wf[:, None, :] + wp[None, :, :] + b_ref[...]


def _build_table(f_table, p_table, fc_w, fc_b):
    t3 = pl.pallas_call(
        _table_body,
        out_shape=jax.ShapeDtypeStruct((_N_F, _N_P, _D), jnp.float32),
    )(
        f_table,
        p_table,
        fc_w[:, :_N_P].T,
        fc_w[:, _N_P:].T,
        fc_b.reshape(1, 1, _D),
    )
    return t3.reshape(_N_F * _N_P * _D)


def _edge_embed(edge, t_flat, n):
    assert n % _CHUNK == 0
    n_chunks = n // _CHUNK
    mesh = plsc.VectorSubcoreMesh(core_axis_name="c", subcore_axis_name="s")

    @functools.partial(
        pl.kernel,
        mesh=mesh,
        compiler_params=pltpu.CompilerParams(
            needs_layout_passes=False, use_tc_tiling_on_sc=True),
        out_type=jax.ShapeDtypeStruct((n, _D), jnp.float32),
        scratch_types=[
            pltpu.VMEM((_CHUNK, _COLS), jnp.float32),
            pltpu.VMEM((_CHUNK, _COLS), jnp.float32),
            pltpu.VMEM((_CHUNK, _COLS), jnp.float32),
            pltpu.VMEM((_CHUNK, _D), jnp.float32),
            pltpu.VMEM((_CHUNK, _D), jnp.float32),
            pltpu.VMEM((_CHUNK, _D), jnp.float32),
            pltpu.VMEM((_N_F * _N_P * _D,), jnp.float32),
            pltpu.SemaphoreType.DMA,
            pltpu.SemaphoreType.DMA,
            pltpu.SemaphoreType.DMA,
            pltpu.SemaphoreType.DMA,
            pltpu.SemaphoreType.DMA,
            pltpu.SemaphoreType.DMA,
        ],
    )
    def run(edge_hbm, t_hbm, out_hbm, ev0, ev1, ev2, ov0, ov1, ov2, t_v,
            si0, si1, si2, so0, so1, so2):
        cid = lax.axis_index("c")
        sid = lax.axis_index("s")
        wid = sid * 2 + cid
        pltpu.sync_copy(t_hbm, t_v)
        my_n = (n_chunks - wid + _N_TILES - 1) // _N_TILES
        edge_bufs = (ev0, ev1, ev2)
        out_bufs = (ov0, ov1, ov2)
        sin = (si0, si1, si2)
        sout = (so0, so1, so2)

        lane = lax.iota(jnp.int32, _LANES)
        fmask = lane < _N_F
        ninf = jnp.full((_LANES,), -jnp.inf, jnp.float32)

        def in_slice(i):
            return edge_hbm.at[pl.ds((wid + i * _N_TILES) * _CHUNK, _CHUNK)]

        def out_slice(i):
            return out_hbm.at[pl.ds((wid + i * _N_TILES) * _CHUNK, _CHUNK)]

        def compute(edge_v, out_v):
            @plsc.parallel_loop(0, _CHUNK, unroll=8)
            def row_body(r):
                va = edge_v[r, pl.ds(0, _LANES)]
                vb = edge_v[r, pl.ds(_N_F, _LANES)]
                va_f = jnp.where(fmask, va, ninf)
                feq = va_f == jnp.max(va_f)
                fidx = plsc.all_reduce_ffs(feq)
                peq = vb == jnp.max(vb)
                pidx = plsc.all_reduce_ffs(peq)
                tb = ((fidx * _N_P + pidx) * _D)[0]
                for q in range(0, _D, _LANES):
                    out_v[r, pl.ds(q, _LANES)] = t_v[pl.ds(tb + q, _LANES)]

        for b in range(3):
            @pl.when(b < my_n)
            def _():
                pltpu.async_copy(in_slice(b), edge_bufs[b], sin[b])

        def outer(j, carry):
            for b in range(3):
                i = 3 * j + b

                @pl.when(i < my_n)
                def _():
                    pltpu.make_async_copy(in_slice(i), edge_bufs[b], sin[b]).wait()

                    @pl.when(i >= 3)
                    def _():
                        pltpu.make_async_copy(
                            out_bufs[b], out_slice(i - 3), sout[b]).wait()

                    compute(edge_bufs[b], out_bufs[b])
                    pltpu.async_copy(out_bufs[b], out_slice(i), sout[b])

                    @pl.when(i + 3 < my_n)
                    def _():
                        pltpu.async_copy(in_slice(i + 3), edge_bufs[b], sin[b])
            return carry

        lax.fori_loop(0, (my_n + 2) // 3, outer, 0)

        for b in range(3):
            @pl.when(b < my_n)
            def _():
                pltpu.make_async_copy(out_bufs[b], out_slice(b), sout[b]).wait()

    return run(edge, t_flat)


def kernel(edge, p_table, f_table, fc_w, fc_b):
    n = edge.shape[0]
    t = _build_table(f_table, p_table, fc_w, fc_b)
    return _edge_embed(edge, t, n)


# speedup vs baseline: 1.8020x; 1.0001x over previous
import functools

import jax
import jax.numpy as jnp
from jax import lax
from jax.experimental import pallas as pl
from jax.experimental.pallas import tpu as pltpu
from jax.experimental.pallas import tpu_sc as plsc

_N_F = 7
_N_P = 16
_COLS = _N_F + _N_P
_D = 64
_CHUNK = 160
_LANES = 16
_N_TILES = 32


def _table_body(f_ref, p_ref, wf_ref, wp_ref, b_ref, t_ref):
    wf = jnp.dot(f_ref[...], wf_ref[...], preferred_element_type=jnp.float32)
    wp = jnp.dot(p_ref[...], wp_ref[...], preferred_element_type=jnp.float32)
    t_ref[...] = wf[:, None, :] + wp[None, :, :] + b_ref[...]


def _build_table(f_table, p_table, fc_w, fc_b):
    t3 = pl.pallas_call(
        _table_body,
        out_shape=jax.ShapeDtypeStruct((_N_F, _N_P, _D), jnp.float32),
    )(
        f_table,
        p_table,
        fc_w[:, :_N_P].T,
        fc_w[:, _N_P:].T,
        fc_b.reshape(1, 1, _D),
    )
    return t3.reshape(_N_F * _N_P * _D)


def _edge_embed(edge, t_flat, n):
    assert n % _CHUNK == 0
    n_chunks = n // _CHUNK
    mesh = plsc.VectorSubcoreMesh(core_axis_name="c", subcore_axis_name="s")

    @functools.partial(
        pl.kernel,
        mesh=mesh,
        compiler_params=pltpu.CompilerParams(
            needs_layout_passes=False, use_tc_tiling_on_sc=True),
        out_type=jax.ShapeDtypeStruct((n, _D), jnp.float32),
        scratch_types=[
            pltpu.VMEM((_CHUNK, _COLS), jnp.float32),
            pltpu.VMEM((_CHUNK, _COLS), jnp.float32),
            pltpu.VMEM((_CHUNK, _COLS), jnp.float32),
            pltpu.VMEM((_CHUNK, _D), jnp.float32),
            pltpu.VMEM((_CHUNK, _D), jnp.float32),
            pltpu.VMEM((_CHUNK, _D), jnp.float32),
            pltpu.VMEM((_N_F * _N_P * _D,), jnp.float32),
            pltpu.SemaphoreType.DMA,
            pltpu.SemaphoreType.DMA,
            pltpu.SemaphoreType.DMA,
            pltpu.SemaphoreType.DMA,
            pltpu.SemaphoreType.DMA,
            pltpu.SemaphoreType.DMA,
        ],
    )
    def run(edge_hbm, t_hbm, out_hbm, ev0, ev1, ev2, ov0, ov1, ov2, t_v,
            si0, si1, si2, so0, so1, so2):
        cid = lax.axis_index("c")
        sid = lax.axis_index("s")
        wid = sid * 2 + cid
        pltpu.sync_copy(t_hbm, t_v)
        my_n = (n_chunks - wid + _N_TILES - 1) // _N_TILES
        edge_bufs = (ev0, ev1, ev2)
        out_bufs = (ov0, ov1, ov2)
        sin = (si0, si1, si2)
        sout = (so0, so1, so2)

        lane = lax.iota(jnp.int32, _LANES)
        fmask = lane < _N_F
        ninf = jnp.full((_LANES,), -jnp.inf, jnp.float32)

        def in_slice(i):
            return edge_hbm.at[pl.ds((wid + i * _N_TILES) * _CHUNK, _CHUNK)]

        def out_slice(i):
            return out_hbm.at[pl.ds((wid + i * _N_TILES) * _CHUNK, _CHUNK)]

        def compute(edge_v, out_v):
            @plsc.parallel_loop(0, _CHUNK, unroll=8)
            def row_body(r):
                va = edge_v[r, pl.ds(0, _LANES)]
                vb = edge_v[r, pl.ds(_N_F, _LANES)]
                va_f = jnp.where(fmask, va, ninf)
                feq = va_f == jnp.max(va_f)
                fidx = plsc.all_reduce_ffs(feq)
                peq = vb == jnp.max(vb)
                pidx = plsc.all_reduce_ffs(peq)
                tb = ((fidx * _N_P + pidx) * _D)[0]
                for q in range(0, _D, _LANES):
                    out_v[r, pl.ds(q, _LANES)] = t_v[pl.ds(tb + q, _LANES)]

        for b in range(3):
            @pl.when(b < my_n)
            def _():
                pltpu.async_copy(in_slice(b), edge_bufs[b], sin[b])

        def outer(j, carry):
            for b in range(3):
                i = 3 * j + b

                @pl.when(i < my_n)
                def _():
                    pltpu.make_async_copy(in_slice(i), edge_bufs[b], sin[b]).wait()

                    @pl.when(i >= 3)
                    def _():
                        pltpu.make_async_copy(
                            out_bufs[b], out_slice(i - 3), sout[b]).wait()

                    compute(edge_bufs[b], out_bufs[b])
                    pltpu.async_copy(out_bufs[b], out_slice(i), sout[b])

                    @pl.when(i + 3 < my_n)
                    def _():
                        pltpu.async_copy(in_slice(i + 3), edge_bufs[b], sin[b])
            return carry

        lax.fori_loop(0, (my_n + 2) // 3, outer, 0)

        for b in range(3):
            @pl.when(b < my_n)
            def _():
                pltpu.make_async_copy(out_bufs[b], out_slice(b), sout[b]).wait()

    return run(edge, t_flat)


def kernel(edge, p_table, f_table, fc_w, fc_b):
    n = edge.shape[0]
    t = _build_table(f_table, p_table, fc_w, fc_b)
    return _edge_embed(edge, t, n)


# speedup vs baseline: 1.8036x; 1.0009x over previous
import functools

import jax
import jax.numpy as jnp
from jax import lax
from jax.experimental import pallas as pl
from jax.experimental.pallas import tpu as pltpu
from jax.experimental.pallas import tpu_sc as plsc

_N_F = 7
_N_P = 16
_COLS = _N_F + _N_P
_D = 64
_CHUNK = 128
_LANES = 16
_N_TILES = 32


def _table_body(f_ref, p_ref, wf_ref, wp_ref, b_ref, t_ref):
    wf = jnp.dot(f_ref[...], wf_ref[...], preferred_element_type=jnp.float32)
    wp = jnp.dot(p_ref[...], wp_ref[...], preferred_element_type=jnp.float32)
    t_ref[...] = wf[:, None, :] + wp[None, :, :] + b_ref[...]


def _build_table(f_table, p_table, fc_w, fc_b):
    t3 = pl.pallas_call(
        _table_body,
        out_shape=jax.ShapeDtypeStruct((_N_F, _N_P, _D), jnp.float32),
    )(
        f_table,
        p_table,
        fc_w[:, :_N_P].T,
        fc_w[:, _N_P:].T,
        fc_b.reshape(1, 1, _D),
    )
    return t3.reshape(_N_F * _N_P * _D)


def _edge_embed(edge, t_flat, n):
    assert n % _CHUNK == 0
    n_chunks = n // _CHUNK
    mesh = plsc.VectorSubcoreMesh(core_axis_name="c", subcore_axis_name="s")

    @functools.partial(
        pl.kernel,
        mesh=mesh,
        compiler_params=pltpu.CompilerParams(
            needs_layout_passes=False, use_tc_tiling_on_sc=True),
        out_type=jax.ShapeDtypeStruct((n, _D), jnp.float32),
        scratch_types=[
            pltpu.VMEM((_CHUNK, _COLS), jnp.float32),
            pltpu.VMEM((_CHUNK, _COLS), jnp.float32),
            pltpu.VMEM((_CHUNK, _COLS), jnp.float32),
            pltpu.VMEM((_CHUNK, _D), jnp.float32),
            pltpu.VMEM((_CHUNK, _D), jnp.float32),
            pltpu.VMEM((_CHUNK, _D), jnp.float32),
            pltpu.VMEM((_N_F * _N_P * _D,), jnp.float32),
            pltpu.SemaphoreType.DMA,
            pltpu.SemaphoreType.DMA,
            pltpu.SemaphoreType.DMA,
            pltpu.SemaphoreType.DMA,
            pltpu.SemaphoreType.DMA,
            pltpu.SemaphoreType.DMA,
        ],
    )
    def run(edge_hbm, t_hbm, out_hbm, ev0, ev1, ev2, ov0, ov1, ov2, t_v,
            si0, si1, si2, so0, so1, so2):
        cid = lax.axis_index("c")
        sid = lax.axis_index("s")
        wid = sid * 2 + cid
        pltpu.sync_copy(t_hbm, t_v)
        my_n = (n_chunks - wid + _N_TILES - 1) // _N_TILES
        edge_bufs = (ev0, ev1, ev2)
        out_bufs = (ov0, ov1, ov2)
        sin = (si0, si1, si2)
        sout = (so0, so1, so2)

        lane = lax.iota(jnp.int32, _LANES)
        fmask = lane < _N_F
        ninf = jnp.full((_LANES,), -jnp.inf, jnp.float32)

        def in_slice(i):
            return edge_hbm.at[pl.ds((wid + i * _N_TILES) * _CHUNK, _CHUNK)]

        def out_slice(i):
            return out_hbm.at[pl.ds((wid + i * _N_TILES) * _CHUNK, _CHUNK)]

        def compute(edge_v, out_v):
            @plsc.parallel_loop(0, _CHUNK, unroll=8)
            def row_body(r):
                va = edge_v[r, pl.ds(0, _LANES)]
                vb = edge_v[r, pl.ds(_N_F, _LANES)]
                va_f = jnp.where(fmask, va, ninf)
                feq = va_f == jnp.max(va_f)
                fidx = plsc.all_reduce_ffs(feq)
                peq = vb == jnp.max(vb)
                pidx = plsc.all_reduce_ffs(peq)
                tb = ((fidx * _N_P + pidx) * _D)[0]
                for q in range(0, _D, _LANES):
                    out_v[r, pl.ds(q, _LANES)] = t_v[pl.ds(tb + q, _LANES)]

        for b in range(3):
            @pl.when(b < my_n)
            def _():
                pltpu.async_copy(in_slice(b), edge_bufs[b], sin[b])

        def outer(j, carry):
            for b in range(3):
                i = 3 * j + b

                @pl.when(i < my_n)
                def _():
                    pltpu.make_async_copy(in_slice(i), edge_bufs[b], sin[b]).wait()

                    @pl.when(i >= 3)
                    def _():
                        pltpu.make_async_copy(
                            out_bufs[b], out_slice(i - 3), sout[b]).wait()

                    compute(edge_bufs[b], out_bufs[b])
                    pltpu.async_copy(out_bufs[b], out_slice(i), sout[b])

                    @pl.when(i + 3 < my_n)
                    def _():
                        pltpu.async_copy(in_slice(i + 3), edge_bufs[b], sin[b])
            return carry

        lax.fori_loop(0, (my_n + 2) // 3, outer, 0)

        for b in range(3):
            @pl.when(b < my_n)
            def _():
                pltpu.make_async_copy(out_bufs[b], out_slice(b), sout[b]).wait()

    return run(edge, t_flat)


def kernel(edge, p_table, f_table, fc_w, fc_b):
    n = edge.shape[0]
    t = _build_table(f_table, p_table, fc_w, fc_b)
    return _edge_embed(edge, t, n)
